# baseline (speedup 1.0000x reference)
import functools

import jax
import jax.numpy as jnp
from jax import lax
from jax.experimental import pallas as pl
from jax.experimental.pallas import tpu as pltpu
from jax.experimental.pallas import tpu_sc as plsc

_K0 = 928981903
_K1 = 3453687069
_KS = (_K0, _K1, _K0 ^ _K1 ^ 0x1BD11BDA)

_B = 128
_V = 100000
_COLS = 2048
_NBT = 38
_VTC = _NBT * _COLS
_W = _V - _VTC
_NW = 32
_W8 = _W // 8
_WIN8 = min(_W8, -(-_W8 // _NW) + 1)
_WROWS = _WIN8 * 8
_ROT = ((13, 15, 26, 6), (17, 29, 16, 24))
_NEG_LN2 = -0.6931471805599453
_LOG2_COEF = (3.81468787113981e-10, 1.442694905424571, -0.721347561295203,
              0.4809185058346664, -0.3606918278810991, 0.28774239323991246,
              -0.239137759619135, 0.2172170985382013, -0.2062543481478371,
              0.12094720695026735)
_SQRT2 = 1.4142135623730951


def _threefry_bits(x1):
    x0 = None
    for g in range(5):
        for r in _ROT[g & 1]:
            x0 = (x1 + jnp.uint32(_KS[0])) if x0 is None else (x0 + x1)
            x1 = ((x1 << jnp.uint32(r)) | (x1 >> jnp.uint32(32 - r))) ^ x0
        x0 = x0 + jnp.uint32(_KS[(g + 1) % 3])
        x1 = x1 + jnp.uint32((_KS[(g + 2) % 3] + g + 1) & 0xFFFFFFFF)
    return x0 ^ x1


def _tc_body(logits_ref, idx_ref, val_ref, runval, runidx, base_s):
    j = pl.program_id(0)

    @pl.when(j == 0)
    def _init():
        runval[...] = jnp.full((_B, 128), -jnp.inf, jnp.float32)
        runidx[...] = jnp.zeros((_B, 128), jnp.int32)
        row = jax.lax.broadcasted_iota(jnp.uint32, (_B, _COLS), 0)
        col = jax.lax.broadcasted_iota(jnp.uint32, (_B, _COLS), 1)
        base_s[...] = row * jnp.uint32(_V) + col + jnp.uint32(_K1)

    x1 = base_s[...] + (j * _COLS).astype(jnp.uint32)
    bits = _threefry_bits(x1)
    fb = (bits >> jnp.uint32(9)) | jnp.uint32(0x3F800000)
    tiny = jnp.float32(jnp.finfo(jnp.float32).tiny)
    u = (pltpu.bitcast(fb, jnp.float32) - jnp.float32(1.0)) + tiny
    g = -jnp.log(-jnp.log(u))
    phi = logits_ref[...] + g
    cidx = jax.lax.broadcasted_iota(jnp.int32, (_B, _COLS), 1) + j * _COLS

    rv = runval[...]
    ri = runidx[...]
    for k in range(_COLS // 128):
        p = phi[:, k * 128:(k + 1) * 128]
        ci = cidx[:, k * 128:(k + 1) * 128]
        upd = p > rv
        rv = jnp.where(upd, p, rv)
        ri = jnp.where(upd, ci, ri)
    runval[...] = rv
    runidx[...] = ri

    @pl.when(j == _NBT - 1)
    def _finish():
        rv2 = runval[...]
        ri2 = runidx[...]
        rowmax = jnp.max(rv2, axis=1, keepdims=True)
        big = jnp.int32(2**31 - 1)
        cand = jnp.where(rv2 == rowmax, ri2, big)
        idx_ref[...] = jnp.min(cand, axis=1, keepdims=True)
        val_ref[...] = rowmax


def _tc_shard(logits_tc):
    return pl.pallas_call(
        _tc_body,
        grid=(_NBT,),
        in_specs=[pl.BlockSpec((_B, _COLS), lambda j: (0, j))],
        out_specs=[
            pl.BlockSpec((_B, 1), lambda j: (0, 0)),
            pl.BlockSpec((_B, 1), lambda j: (0, 0)),
        ],
        out_shape=[
            jax.ShapeDtypeStruct((_B, 1), jnp.int32),
            jax.ShapeDtypeStruct((_B, 1), jnp.float32),
        ],
        scratch_shapes=[
            pltpu.VMEM((_B, 128), jnp.float32),
            pltpu.VMEM((_B, 128), jnp.int32),
            pltpu.VMEM((_B, _COLS), jnp.uint32),
        ],
    )(logits_tc)


def _neg_log(x):
    xb = lax.bitcast_convert_type(x, jnp.uint32)
    e = (xb >> jnp.uint32(23)).astype(jnp.int32) - jnp.int32(127)
    y = lax.bitcast_convert_type(
        (xb & jnp.uint32(0x7FFFFF)) | jnp.uint32(0x3F800000), jnp.float32)
    big = y >= jnp.float32(_SQRT2)
    y = jnp.where(big, y * jnp.float32(0.5), y)
    e = jnp.where(big, e + jnp.int32(1), e)
    z = y - jnp.float32(1.0)
    acc = jnp.full((16,), _LOG2_COEF[-1], jnp.float32)
    for c in reversed(_LOG2_COEF[:-1]):
        acc = acc * z + jnp.float32(c)
    log2x = e.astype(jnp.float32) + acc
    return log2x * jnp.float32(_NEG_LN2)


def _sc_kernel_body(lt_hbm, val_hbm, idx_hbm, buf, vout, iout, dsem):
    c = lax.axis_index("c")
    s = lax.axis_index("s")
    w = s * 2 + c
    s8 = (w * (_W8 - _WIN8)) // (_NW - 1)
    vstart = _VTC + s8 * 8
    pltpu.async_copy(lt_hbm.at[pl.ds(vstart, _WROWS)], buf, dsem).wait()

    tiny = jnp.float32(jnp.finfo(jnp.float32).tiny)
    binit = [jnp.full((16,), -jnp.inf, jnp.float32) for _ in range(8)]
    iinit = [jnp.zeros((16,), jnp.int32) for _ in range(8)]
    lane_bvs = [
        jnp.uint32(_V) * (jnp.uint32(g * 16)
                          + jax.lax.broadcasted_iota(jnp.uint32, (16,), 0))
        for g in range(8)
    ]

    def body(r, carry):
        bvs, bis = carry
        vglob = vstart + r
        cbase = vglob.astype(jnp.uint32) + jnp.uint32(_K1)
        nbvs, nbis = [], []
        for g in range(8):
            x1 = lane_bvs[g] + cbase
            bits = _threefry_bits(x1)
            fb = (bits >> jnp.uint32(9)) | jnp.uint32(0x3F800000)
            u = (lax.bitcast_convert_type(fb, jnp.float32)
                 - jnp.float32(1.0)) + tiny
            gum = _neg_log(_neg_log(u))
            phi = buf[r, pl.ds(g * 16, 16)] + gum
            upd = phi > bvs[g]
            nbvs.append(jnp.where(upd, phi, bvs[g]))
            nbis.append(jnp.where(upd, jnp.zeros((16,), jnp.int32) + vglob,
                                  bis[g]))
        return nbvs, nbis

    bvs, bis = lax.fori_loop(0, _WROWS, body, (binit, iinit))
    for g in range(8):
        vout[pl.ds(g * 16, 16)] = bvs[g]
        iout[pl.ds(g * 16, 16)] = bis[g]
    pltpu.sync_copy(vout, val_hbm.at[w])
    pltpu.sync_copy(iout, idx_hbm.at[w])


@functools.partial(
    pl.kernel,
    out_type=[
        jax.ShapeDtypeStruct((_NW, _B), jnp.float32),
        jax.ShapeDtypeStruct((_NW, _B), jnp.int32),
    ],
    mesh=plsc.VectorSubcoreMesh(core_axis_name="c", subcore_axis_name="s"),
    scratch_types=[
        pltpu.VMEM((_WROWS, _B), jnp.float32),
        pltpu.VMEM((_B,), jnp.float32),
        pltpu.VMEM((_B,), jnp.int32),
        pltpu.SemaphoreType.DMA,
    ],
)
def _sc_shard(lt_hbm, val_hbm, idx_hbm, buf, vout, iout, dsem):
    _sc_kernel_body(lt_hbm, val_hbm, idx_hbm, buf, vout, iout, dsem)


def _merge_body(scv_ref, sci_ref, tci_ref, tcv_ref, out_ref):
    scv = scv_ref[...].T
    sci = sci_ref[...].T
    scm = jnp.max(scv, axis=1, keepdims=True)
    big = jnp.int32(2**31 - 1)
    cand = jnp.where(scv == scm, sci, big)
    scbi = jnp.min(cand, axis=1, keepdims=True)
    take = scm > tcv_ref[...]
    out_ref[...] = jnp.where(take, scbi, tci_ref[...])


def _merge(scval, scidx, tcidx, tcval):
    return pl.pallas_call(
        _merge_body,
        out_shape=jax.ShapeDtypeStruct((_B, 1), jnp.int32),
    )(scval, scidx, tcidx, tcval)


@jax.jit
def kernel(logits):
    lt = logits.T
    scval, scidx = _sc_shard(lt)
    tcidx, tcval = _tc_shard(lt[:_VTC].T)
    out = _merge(scval, scidx, tcidx, tcval)
    return out.reshape(_B)


# speedup vs baseline: 1.1653x; 1.1653x over previous
import functools

import jax
import jax.numpy as jnp
from jax import lax
from jax.experimental import pallas as pl
from jax.experimental.pallas import tpu as pltpu
from jax.experimental.pallas import tpu_sc as plsc

_K0 = 928981903
_K1 = 3453687069
_KS = (_K0, _K1, _K0 ^ _K1 ^ 0x1BD11BDA)

_B = 128
_V = 100000
_COLS = 2048
_NBT = 38
_VTC = _NBT * _COLS
_W = _V - _VTC
_NW = 32
_W8 = _W // 8
_WIN8 = min(_W8, -(-_W8 // _NW) + 1)
_WROWS = _WIN8 * 8
_ROT = ((13, 15, 26, 6), (17, 29, 16, 24))
_NEG_LN2 = -0.6931471805599453
_LOG2_COEF = (3.81468787113981e-10, 1.442694905424571, -0.721347561295203,
              0.4809185058346664, -0.3606918278810991, 0.28774239323991246,
              -0.239137759619135, 0.2172170985382013, -0.2062543481478371,
              0.12094720695026735)
_SQRT2 = 1.4142135623730951


def _threefry_bits(x1):
    x0 = None
    for g in range(5):
        for r in _ROT[g & 1]:
            x0 = (x1 + jnp.uint32(_KS[0])) if x0 is None else (x0 + x1)
            x1 = ((x1 << jnp.uint32(r)) | (x1 >> jnp.uint32(32 - r))) ^ x0
        x0 = x0 + jnp.uint32(_KS[(g + 1) % 3])
        x1 = x1 + jnp.uint32((_KS[(g + 2) % 3] + g + 1) & 0xFFFFFFFF)
    return x0 ^ x1


def _tc_body(logits_ref, idx_ref, val_ref, runval, runidx, base_s):
    j = pl.program_id(0)

    @pl.when(j == 0)
    def _init():
        runval[...] = jnp.full((_B, 128), -jnp.inf, jnp.float32)
        runidx[...] = jnp.zeros((_B, 128), jnp.int32)
        row = jax.lax.broadcasted_iota(jnp.uint32, (_B, _COLS), 0)
        col = jax.lax.broadcasted_iota(jnp.uint32, (_B, _COLS), 1)
        base_s[...] = row * jnp.uint32(_V) + col + jnp.uint32(_K1)

    x1 = base_s[...] + (j * _COLS).astype(jnp.uint32)
    bits = _threefry_bits(x1)
    fb = (bits >> jnp.uint32(9)) | jnp.uint32(0x3F800000)
    tiny = jnp.float32(jnp.finfo(jnp.float32).tiny)
    u = (pltpu.bitcast(fb, jnp.float32) - jnp.float32(1.0)) + tiny
    g = -jnp.log(-jnp.log(u))
    phi = logits_ref[...] + g
    cidx = jax.lax.broadcasted_iota(jnp.int32, (_B, _COLS), 1) + j * _COLS

    rv = runval[...]
    ri = runidx[...]
    for k in range(_COLS // 128):
        p = phi[:, k * 128:(k + 1) * 128]
        ci = cidx[:, k * 128:(k + 1) * 128]
        upd = p > rv
        rv = jnp.where(upd, p, rv)
        ri = jnp.where(upd, ci, ri)
    runval[...] = rv
    runidx[...] = ri

    @pl.when(j == _NBT - 1)
    def _finish():
        rv2 = runval[...]
        ri2 = runidx[...]
        rowmax = jnp.max(rv2, axis=1, keepdims=True)
        big = jnp.int32(2**31 - 1)
        cand = jnp.where(rv2 == rowmax, ri2, big)
        idx_ref[...] = jnp.min(cand, axis=1, keepdims=True)
        val_ref[...] = rowmax


def _tc_shard(logits_tc):
    return pl.pallas_call(
        _tc_body,
        grid=(_NBT,),
        in_specs=[pl.BlockSpec((_B, _COLS), lambda j: (0, j))],
        out_specs=[
            pl.BlockSpec((_B, 1), lambda j: (0, 0)),
            pl.BlockSpec((_B, 1), lambda j: (0, 0)),
        ],
        out_shape=[
            jax.ShapeDtypeStruct((_B, 1), jnp.int32),
            jax.ShapeDtypeStruct((_B, 1), jnp.float32),
        ],
        scratch_shapes=[
            pltpu.VMEM((_B, 128), jnp.float32),
            pltpu.VMEM((_B, 128), jnp.int32),
            pltpu.VMEM((_B, _COLS), jnp.uint32),
        ],
    )(logits_tc)


def _neg_log(x):
    xb = lax.bitcast_convert_type(x, jnp.uint32)
    e = (xb >> jnp.uint32(23)).astype(jnp.int32) - jnp.int32(127)
    y = lax.bitcast_convert_type(
        (xb & jnp.uint32(0x7FFFFF)) | jnp.uint32(0x3F800000), jnp.float32)
    big = y >= jnp.float32(_SQRT2)
    y = jnp.where(big, y * jnp.float32(0.5), y)
    e = jnp.where(big, e + jnp.int32(1), e)
    z = y - jnp.float32(1.0)
    acc = jnp.full((16,), _LOG2_COEF[-1], jnp.float32)
    for c in reversed(_LOG2_COEF[:-1]):
        acc = acc * z + jnp.float32(c)
    log2x = e.astype(jnp.float32) + acc
    return log2x * jnp.float32(_NEG_LN2)


def _sc_kernel_body(lt_hbm, val_hbm, idx_hbm, buf, vout, iout, dsem):
    c = lax.axis_index("c")
    s = lax.axis_index("s")
    w = s * 2 + c
    s8 = (w * (_W8 - _WIN8)) // (_NW - 1)
    vstart = _VTC + s8 * 8
    pltpu.async_copy(lt_hbm.at[pl.ds(vstart, _WROWS)], buf, dsem).wait()

    tiny = jnp.float32(jnp.finfo(jnp.float32).tiny)
    binit = [jnp.full((16,), -jnp.inf, jnp.float32) for _ in range(8)]
    iinit = [jnp.zeros((16,), jnp.int32) for _ in range(8)]
    lane_bvs = [
        jnp.uint32(_V) * (jnp.uint32(g * 16)
                          + jax.lax.broadcasted_iota(jnp.uint32, (16,), 0))
        for g in range(8)
    ]

    def body(r, carry):
        bvs, bis = carry
        vglob = vstart + r
        cbase = vglob.astype(jnp.uint32) + jnp.uint32(_K1)
        nbvs, nbis = [], []
        for g in range(8):
            x1 = lane_bvs[g] + cbase
            bits = _threefry_bits(x1)
            fb = (bits >> jnp.uint32(9)) | jnp.uint32(0x3F800000)
            u = (lax.bitcast_convert_type(fb, jnp.float32)
                 - jnp.float32(1.0)) + tiny
            gum = _neg_log(_neg_log(u))
            phi = buf[r, pl.ds(g * 16, 16)] + gum
            upd = phi > bvs[g]
            nbvs.append(jnp.where(upd, phi, bvs[g]))
            nbis.append(jnp.where(upd, jnp.zeros((16,), jnp.int32) + vglob,
                                  bis[g]))
        return nbvs, nbis

    bvs, bis = lax.fori_loop(0, _WROWS, body, (binit, iinit))
    for g in range(8):
        vout[pl.ds(g * 16, 16)] = bvs[g]
        iout[pl.ds(g * 16, 16)] = bis[g]
    pltpu.sync_copy(vout, val_hbm.at[w])
    pltpu.sync_copy(iout, idx_hbm.at[w])


@functools.partial(
    pl.kernel,
    out_type=[
        jax.ShapeDtypeStruct((_NW, _B), jnp.float32),
        jax.ShapeDtypeStruct((_NW, _B), jnp.int32),
    ],
    mesh=plsc.VectorSubcoreMesh(core_axis_name="c", subcore_axis_name="s"),
    scratch_types=[
        pltpu.VMEM((_WROWS, _B), jnp.float32),
        pltpu.VMEM((_B,), jnp.float32),
        pltpu.VMEM((_B,), jnp.int32),
        pltpu.SemaphoreType.DMA,
    ],
)
def _sc_shard(lt_hbm, val_hbm, idx_hbm, buf, vout, iout, dsem):
    _sc_kernel_body(lt_hbm, val_hbm, idx_hbm, buf, vout, iout, dsem)


def _merge_body(scv_ref, sci_ref, tci_ref, tcv_ref, out_ref):
    scv = scv_ref[...].T
    sci = sci_ref[...].T
    scm = jnp.max(scv, axis=1, keepdims=True)
    big = jnp.int32(2**31 - 1)
    cand = jnp.where(scv == scm, sci, big)
    scbi = jnp.min(cand, axis=1, keepdims=True)
    take = scm > tcv_ref[...]
    out_ref[...] = jnp.where(take, scbi, tci_ref[...])


def _merge(scval, scidx, tcidx, tcval):
    return pl.pallas_call(
        _merge_body,
        out_shape=jax.ShapeDtypeStruct((_B, 1), jnp.int32),
    )(scval, scidx, tcidx, tcval)


@jax.jit
def kernel(logits):
    lt = logits.T
    scval, scidx = _sc_shard(lt)
    tcidx, tcval = _tc_shard(logits)
    out = _merge(scval, scidx, tcidx, tcval)
    return out.reshape(_B)


# speedup vs baseline: 1.1668x; 1.0013x over previous
import functools

import jax
import jax.numpy as jnp
from jax import lax
from jax.experimental import pallas as pl
from jax.experimental.pallas import tpu as pltpu
from jax.experimental.pallas import tpu_sc as plsc

_K0 = 928981903
_K1 = 3453687069
_KS = (_K0, _K1, _K0 ^ _K1 ^ 0x1BD11BDA)

_B = 128
_V = 100000
_COLS = 2048
_NBT = 38
_VTC = _NBT * _COLS
_W = _V - _VTC
_NW = 32
_W8 = _W // 8
_WIN8 = min(_W8, -(-_W8 // _NW) + 1)
_WROWS = _WIN8 * 8
_ROT = ((13, 15, 26, 6), (17, 29, 16, 24))
_NEG_LN2 = -0.6931471805599453
_LOG2_COEF = (3.81468787113981e-10, 1.442694905424571, -0.721347561295203,
              0.4809185058346664, -0.3606918278810991, 0.28774239323991246,
              -0.239137759619135, 0.2172170985382013, -0.2062543481478371,
              0.12094720695026735)
_SQRT2 = 1.4142135623730951


def _threefry_bits(x1):
    x0 = None
    for g in range(5):
        for r in _ROT[g & 1]:
            x0 = (x1 + jnp.uint32(_KS[0])) if x0 is None else (x0 + x1)
            x1 = ((x1 << jnp.uint32(r)) | (x1 >> jnp.uint32(32 - r))) ^ x0
        x0 = x0 + jnp.uint32(_KS[(g + 1) % 3])
        x1 = x1 + jnp.uint32((_KS[(g + 2) % 3] + g + 1) & 0xFFFFFFFF)
    return x0 ^ x1


def _tc_body(logits_ref, idx_ref, val_ref, runval, runidx, base_s):
    j = pl.program_id(0)

    @pl.when(j == 0)
    def _init():
        runval[...] = jnp.full((_B, 128), -jnp.inf, jnp.float32)
        runidx[...] = jnp.zeros((_B, 128), jnp.int32)
        row = jax.lax.broadcasted_iota(jnp.uint32, (_B, _COLS), 0)
        col = jax.lax.broadcasted_iota(jnp.uint32, (_B, _COLS), 1)
        base_s[...] = row * jnp.uint32(_V) + col + jnp.uint32(_K1)

    x1 = base_s[...] + (j * _COLS).astype(jnp.uint32)
    bits = _threefry_bits(x1)
    fb = (bits >> jnp.uint32(9)) | jnp.uint32(0x3F800000)
    tiny = jnp.float32(jnp.finfo(jnp.float32).tiny)
    u = (pltpu.bitcast(fb, jnp.float32) - jnp.float32(1.0)) + tiny
    g = -jnp.log(-jnp.log(u))
    phi = logits_ref[...] + g
    cidx = jax.lax.broadcasted_iota(jnp.int32, (_B, _COLS), 1) + j * _COLS

    rv = runval[...]
    ri = runidx[...]
    for k in range(_COLS // 128):
        p = phi[:, k * 128:(k + 1) * 128]
        ci = cidx[:, k * 128:(k + 1) * 128]
        upd = p > rv
        rv = jnp.where(upd, p, rv)
        ri = jnp.where(upd, ci, ri)
    runval[...] = rv
    runidx[...] = ri

    @pl.when(j == _NBT - 1)
    def _finish():
        rv2 = runval[...]
        ri2 = runidx[...]
        rowmax = jnp.max(rv2, axis=1, keepdims=True)
        big = jnp.int32(2**31 - 1)
        cand = jnp.where(rv2 == rowmax, ri2, big)
        idx_ref[...] = jnp.min(cand, axis=1, keepdims=True)
        val_ref[...] = rowmax


def _tc_shard(logits_tc):
    return pl.pallas_call(
        _tc_body,
        grid=(_NBT,),
        in_specs=[pl.BlockSpec((_B, _COLS), lambda j: (0, j))],
        out_specs=[
            pl.BlockSpec((_B, 1), lambda j: (0, 0)),
            pl.BlockSpec((_B, 1), lambda j: (0, 0)),
        ],
        out_shape=[
            jax.ShapeDtypeStruct((_B, 1), jnp.int32),
            jax.ShapeDtypeStruct((_B, 1), jnp.float32),
        ],
        scratch_shapes=[
            pltpu.VMEM((_B, 128), jnp.float32),
            pltpu.VMEM((_B, 128), jnp.int32),
            pltpu.VMEM((_B, _COLS), jnp.uint32),
        ],
    )(logits_tc)


def _neg_log(x):
    xb = lax.bitcast_convert_type(x, jnp.uint32)
    e = (xb >> jnp.uint32(23)).astype(jnp.int32) - jnp.int32(127)
    y = lax.bitcast_convert_type(
        (xb & jnp.uint32(0x7FFFFF)) | jnp.uint32(0x3F800000), jnp.float32)
    big = y >= jnp.float32(_SQRT2)
    y = jnp.where(big, y * jnp.float32(0.5), y)
    e = jnp.where(big, e + jnp.int32(1), e)
    z = y - jnp.float32(1.0)
    acc = jnp.full((16,), _LOG2_COEF[-1], jnp.float32)
    for c in reversed(_LOG2_COEF[:-1]):
        acc = acc * z + jnp.float32(c)
    log2x = e.astype(jnp.float32) + acc
    return log2x * jnp.float32(_NEG_LN2)


def _sc_kernel_body(lt_hbm, val_hbm, idx_hbm, buf, vout, iout, dsem):
    c = lax.axis_index("c")
    s = lax.axis_index("s")
    w = s * 2 + c
    s8 = (w * (_W8 - _WIN8)) // (_NW - 1)
    vstart = _VTC + s8 * 8
    pltpu.async_copy(lt_hbm.at[pl.ds(vstart, _WROWS)], buf, dsem).wait()

    tiny = jnp.float32(jnp.finfo(jnp.float32).tiny)
    binit = [jnp.full((16,), -jnp.inf, jnp.float32) for _ in range(8)]
    iinit = [jnp.zeros((16,), jnp.int32) for _ in range(8)]
    lane_bvs = [
        jnp.uint32(_V) * (jnp.uint32(g * 16)
                          + jax.lax.broadcasted_iota(jnp.uint32, (16,), 0))
        for g in range(8)
    ]

    def body(r, carry):
        bvs, bis = carry
        vglob = vstart + r
        cbase = vglob.astype(jnp.uint32) + jnp.uint32(_K1)
        nbvs, nbis = [], []
        for g in range(8):
            x1 = lane_bvs[g] + cbase
            bits = _threefry_bits(x1)
            fb = (bits >> jnp.uint32(9)) | jnp.uint32(0x3F800000)
            u = (lax.bitcast_convert_type(fb, jnp.float32)
                 - jnp.float32(1.0)) + tiny
            gum = _neg_log(_neg_log(u))
            phi = buf[r, pl.ds(g * 16, 16)] + gum
            upd = phi > bvs[g]
            nbvs.append(jnp.where(upd, phi, bvs[g]))
            nbis.append(jnp.where(upd, jnp.zeros((16,), jnp.int32) + vglob,
                                  bis[g]))
        return nbvs, nbis

    bvs, bis = plsc.parallel_loop(
        0, _WROWS, unroll=2, carry=(binit, iinit))(body)
    for g in range(8):
        vout[pl.ds(g * 16, 16)] = bvs[g]
        iout[pl.ds(g * 16, 16)] = bis[g]
    pltpu.sync_copy(vout, val_hbm.at[w])
    pltpu.sync_copy(iout, idx_hbm.at[w])


@functools.partial(
    pl.kernel,
    out_type=[
        jax.ShapeDtypeStruct((_NW, _B), jnp.float32),
        jax.ShapeDtypeStruct((_NW, _B), jnp.int32),
    ],
    mesh=plsc.VectorSubcoreMesh(core_axis_name="c", subcore_axis_name="s"),
    scratch_types=[
        pltpu.VMEM((_WROWS, _B), jnp.float32),
        pltpu.VMEM((_B,), jnp.float32),
        pltpu.VMEM((_B,), jnp.int32),
        pltpu.SemaphoreType.DMA,
    ],
)
def _sc_shard(lt_hbm, val_hbm, idx_hbm, buf, vout, iout, dsem):
    _sc_kernel_body(lt_hbm, val_hbm, idx_hbm, buf, vout, iout, dsem)


def _merge_body(scv_ref, sci_ref, tci_ref, tcv_ref, out_ref):
    scv = scv_ref[...].T
    sci = sci_ref[...].T
    scm = jnp.max(scv, axis=1, keepdims=True)
    big = jnp.int32(2**31 - 1)
    cand = jnp.where(scv == scm, sci, big)
    scbi = jnp.min(cand, axis=1, keepdims=True)
    take = scm > tcv_ref[...]
    out_ref[...] = jnp.where(take, scbi, tci_ref[...])


def _merge(scval, scidx, tcidx, tcval):
    return pl.pallas_call(
        _merge_body,
        out_shape=jax.ShapeDtypeStruct((_B, 1), jnp.int32),
    )(scval, scidx, tcidx, tcval)


@jax.jit
def kernel(logits):
    lt = logits.T
    scval, scidx = _sc_shard(lt)
    tcidx, tcval = _tc_shard(logits)
    out = _merge(scval, scidx, tcidx, tcval)
    return out.reshape(_B)


# speedup vs baseline: 1.1670x; 1.0002x over previous
import functools

import jax
import jax.numpy as jnp
from jax import lax
from jax.experimental import pallas as pl
from jax.experimental.pallas import tpu as pltpu
from jax.experimental.pallas import tpu_sc as plsc

_K0 = 928981903
_K1 = 3453687069
_KS = (_K0, _K1, _K0 ^ _K1 ^ 0x1BD11BDA)

_B = 128
_V = 100000
_COLS = 2048
_NBT = 38
_VTC = _NBT * _COLS
_W = _V - _VTC
_NW = 32
_W8 = _W // 8
_WIN8 = min(_W8, -(-_W8 // _NW) + 1)
_WROWS = _WIN8 * 8
_ROT = ((13, 15, 26, 6), (17, 29, 16, 24))
_NEG_LN2 = -0.6931471805599453
_LOG2_COEF = (3.81468787113981e-10, 1.442694905424571, -0.721347561295203,
              0.4809185058346664, -0.3606918278810991, 0.28774239323991246,
              -0.239137759619135, 0.2172170985382013, -0.2062543481478371,
              0.12094720695026735)
_SQRT2 = 1.4142135623730951


def _threefry_bits(x1):
    x0 = None
    for g in range(5):
        for r in _ROT[g & 1]:
            x0 = (x1 + jnp.uint32(_KS[0])) if x0 is None else (x0 + x1)
            x1 = ((x1 << jnp.uint32(r)) | (x1 >> jnp.uint32(32 - r))) ^ x0
        x0 = x0 + jnp.uint32(_KS[(g + 1) % 3])
        x1 = x1 + jnp.uint32((_KS[(g + 2) % 3] + g + 1) & 0xFFFFFFFF)
    return x0 ^ x1


def _tc_body(logits_ref, idx_ref, val_ref, runval, runidx, base_s):
    j = pl.program_id(0)

    @pl.when(j == 0)
    def _init():
        runval[...] = jnp.full((_B, 128), -jnp.inf, jnp.float32)
        runidx[...] = jnp.zeros((_B, 128), jnp.int32)
        row = jax.lax.broadcasted_iota(jnp.uint32, (_B, _COLS), 0)
        col = jax.lax.broadcasted_iota(jnp.uint32, (_B, _COLS), 1)
        base_s[...] = row * jnp.uint32(_V) + col + jnp.uint32(_K1)

    x1 = base_s[...] + (j * _COLS).astype(jnp.uint32)
    bits = _threefry_bits(x1)
    fb = (bits >> jnp.uint32(9)) | jnp.uint32(0x3F800000)
    tiny = jnp.float32(jnp.finfo(jnp.float32).tiny)
    u = (pltpu.bitcast(fb, jnp.float32) - jnp.float32(1.0)) + tiny
    g = -jnp.log(-jnp.log(u))
    phi = logits_ref[...] + g
    cidx = jax.lax.broadcasted_iota(jnp.int32, (_B, _COLS), 1) + j * _COLS

    rv = runval[...]
    ri = runidx[...]
    for k in range(_COLS // 128):
        p = phi[:, k * 128:(k + 1) * 128]
        ci = cidx[:, k * 128:(k + 1) * 128]
        upd = p > rv
        rv = jnp.where(upd, p, rv)
        ri = jnp.where(upd, ci, ri)
    runval[...] = rv
    runidx[...] = ri

    @pl.when(j == _NBT - 1)
    def _finish():
        rv2 = runval[...]
        ri2 = runidx[...]
        rowmax = jnp.max(rv2, axis=1, keepdims=True)
        big = jnp.int32(2**31 - 1)
        cand = jnp.where(rv2 == rowmax, ri2, big)
        idx_ref[...] = jnp.min(cand, axis=1, keepdims=True)
        val_ref[...] = rowmax


def _tc_shard(logits_tc):
    return pl.pallas_call(
        _tc_body,
        grid=(_NBT,),
        in_specs=[pl.BlockSpec((_B, _COLS), lambda j: (0, j))],
        out_specs=[
            pl.BlockSpec((_B, 1), lambda j: (0, 0)),
            pl.BlockSpec((_B, 1), lambda j: (0, 0)),
        ],
        out_shape=[
            jax.ShapeDtypeStruct((_B, 1), jnp.int32),
            jax.ShapeDtypeStruct((_B, 1), jnp.float32),
        ],
        scratch_shapes=[
            pltpu.VMEM((_B, 128), jnp.float32),
            pltpu.VMEM((_B, 128), jnp.int32),
            pltpu.VMEM((_B, _COLS), jnp.uint32),
        ],
    )(logits_tc)


def _neg_log(x):
    xb = lax.bitcast_convert_type(x, jnp.uint32)
    e = (xb >> jnp.uint32(23)).astype(jnp.int32) - jnp.int32(127)
    y = lax.bitcast_convert_type(
        (xb & jnp.uint32(0x7FFFFF)) | jnp.uint32(0x3F800000), jnp.float32)
    big = y >= jnp.float32(_SQRT2)
    y = jnp.where(big, y * jnp.float32(0.5), y)
    e = jnp.where(big, e + jnp.int32(1), e)
    z = y - jnp.float32(1.0)
    acc = jnp.full((16,), _LOG2_COEF[-1], jnp.float32)
    for c in reversed(_LOG2_COEF[:-1]):
        acc = acc * z + jnp.float32(c)
    log2x = e.astype(jnp.float32) + acc
    return log2x * jnp.float32(_NEG_LN2)


def _sc_kernel_body(lt_hbm, val_hbm, idx_hbm, buf, vout, iout, dsem):
    c = lax.axis_index("c")
    s = lax.axis_index("s")
    w = s * 2 + c
    s8 = (w * (_W8 - _WIN8)) // (_NW - 1)
    vstart = _VTC + s8 * 8
    pltpu.async_copy(lt_hbm.at[pl.ds(vstart, _WROWS)], buf, dsem).wait()

    tiny = jnp.float32(jnp.finfo(jnp.float32).tiny)
    binit = [jnp.full((16,), -jnp.inf, jnp.float32) for _ in range(8)]
    iinit = [jnp.zeros((16,), jnp.int32) for _ in range(8)]
    lane_bvs = [
        jnp.uint32(_V) * (jnp.uint32(g * 16)
                          + jax.lax.broadcasted_iota(jnp.uint32, (16,), 0))
        for g in range(8)
    ]

    def body(r, carry):
        bvs, bis = carry
        vglob = vstart + r
        cbase = vglob.astype(jnp.uint32) + jnp.uint32(_K1)
        nbvs, nbis = [], []
        for g in range(8):
            x1 = lane_bvs[g] + cbase
            bits = _threefry_bits(x1)
            fb = (bits >> jnp.uint32(9)) | jnp.uint32(0x3F800000)
            u = (lax.bitcast_convert_type(fb, jnp.float32)
                 - jnp.float32(1.0)) + tiny
            gum = _neg_log(_neg_log(u))
            phi = buf[r, pl.ds(g * 16, 16)] + gum
            upd = phi > bvs[g]
            nbvs.append(jnp.where(upd, phi, bvs[g]))
            nbis.append(jnp.where(upd, jnp.zeros((16,), jnp.int32) + vglob,
                                  bis[g]))
        return nbvs, nbis

    bvs, bis = plsc.parallel_loop(
        0, _WROWS, unroll=4, carry=(binit, iinit))(body)
    for g in range(8):
        vout[pl.ds(g * 16, 16)] = bvs[g]
        iout[pl.ds(g * 16, 16)] = bis[g]
    pltpu.sync_copy(vout, val_hbm.at[w])
    pltpu.sync_copy(iout, idx_hbm.at[w])


@functools.partial(
    pl.kernel,
    out_type=[
        jax.ShapeDtypeStruct((_NW, _B), jnp.float32),
        jax.ShapeDtypeStruct((_NW, _B), jnp.int32),
    ],
    mesh=plsc.VectorSubcoreMesh(core_axis_name="c", subcore_axis_name="s"),
    scratch_types=[
        pltpu.VMEM((_WROWS, _B), jnp.float32),
        pltpu.VMEM((_B,), jnp.float32),
        pltpu.VMEM((_B,), jnp.int32),
        pltpu.SemaphoreType.DMA,
    ],
)
def _sc_shard(lt_hbm, val_hbm, idx_hbm, buf, vout, iout, dsem):
    _sc_kernel_body(lt_hbm, val_hbm, idx_hbm, buf, vout, iout, dsem)


def _merge_body(scv_ref, sci_ref, tci_ref, tcv_ref, out_ref):
    scv = scv_ref[...].T
    sci = sci_ref[...].T
    scm = jnp.max(scv, axis=1, keepdims=True)
    big = jnp.int32(2**31 - 1)
    cand = jnp.where(scv == scm, sci, big)
    scbi = jnp.min(cand, axis=1, keepdims=True)
    take = scm > tcv_ref[...]
    out_ref[...] = jnp.where(take, scbi, tci_ref[...])


def _merge(scval, scidx, tcidx, tcval):
    return pl.pallas_call(
        _merge_body,
        out_shape=jax.ShapeDtypeStruct((_B, 1), jnp.int32),
    )(scval, scidx, tcidx, tcval)


@jax.jit
def kernel(logits):
    lt = logits.T
    scval, scidx = _sc_shard(lt)
    tcidx, tcval = _tc_shard(logits)
    out = _merge(scval, scidx, tcidx, tcval)
    return out.reshape(_B)


# speedup vs baseline: 1.2179x; 1.0437x over previous
import functools

import jax
import jax.numpy as jnp
from jax import lax
from jax.experimental import pallas as pl
from jax.experimental.pallas import tpu as pltpu
from jax.experimental.pallas import tpu_sc as plsc

_K0 = 928981903
_K1 = 3453687069
_KS = (_K0, _K1, _K0 ^ _K1 ^ 0x1BD11BDA)

_B = 128
_V = 100000
_COLS = 2048
_NBT = 38
_VTC = _NBT * _COLS
_W = _V - _VTC
_NW = 32
_W8 = _W // 8
_WIN8 = min(_W8, -(-_W8 // _NW) + 1)
_WROWS = _WIN8 * 8
_ROT = ((13, 15, 26, 6), (17, 29, 16, 24))
_NEG_LN2 = -0.6931471805599453
_LOG2_COEF = (3.81468787113981e-10, 1.442694905424571, -0.721347561295203,
              0.4809185058346664, -0.3606918278810991, 0.28774239323991246,
              -0.239137759619135, 0.2172170985382013, -0.2062543481478371,
              0.12094720695026735)
_SQRT2 = 1.4142135623730951


def _threefry_bits(x1):
    x0 = None
    for g in range(5):
        for r in _ROT[g & 1]:
            x0 = (x1 + jnp.uint32(_KS[0])) if x0 is None else (x0 + x1)
            x1 = ((x1 << jnp.uint32(r)) | (x1 >> jnp.uint32(32 - r))) ^ x0
        x0 = x0 + jnp.uint32(_KS[(g + 1) % 3])
        x1 = x1 + jnp.uint32((_KS[(g + 2) % 3] + g + 1) & 0xFFFFFFFF)
    return x0 ^ x1


def _tc_body(logits_ref, idx_ref, val_ref, runval, runidx, base_s, eye_s):
    j = pl.program_id(0)

    @pl.when(j == 0)
    def _init():
        runval[...] = jnp.full((_B, 128), -jnp.inf, jnp.float32)
        runidx[...] = jnp.zeros((_B, 128), jnp.int32)
        row = jax.lax.broadcasted_iota(jnp.uint32, (_B, _COLS), 0)
        col = jax.lax.broadcasted_iota(jnp.uint32, (_B, _COLS), 1)
        base_s[...] = row * jnp.uint32(_V) + col + jnp.uint32(_K1)
        r8 = jax.lax.broadcasted_iota(jnp.int32, (_B, _B), 0)
        c8 = jax.lax.broadcasted_iota(jnp.int32, (_B, _B), 1)
        eye_s[...] = jnp.where(r8 == c8, jnp.float32(1.0), jnp.float32(0.0))

    x1 = base_s[...] + (j * _COLS).astype(jnp.uint32)
    bits = _threefry_bits(x1)
    fb = (bits >> jnp.uint32(9)) | jnp.uint32(0x3F800000)
    tiny = jnp.float32(jnp.finfo(jnp.float32).tiny)
    u = (pltpu.bitcast(fb, jnp.float32) - jnp.float32(1.0)) + tiny
    g = -jnp.log(-jnp.log(u))
    cidx = jax.lax.broadcasted_iota(jnp.int32, (_B, _COLS), 1) + j * _COLS

    eye = eye_s[...]
    rv = runval[...]
    ri = runidx[...]
    for k in range(_COLS // 128):
        chunk = logits_ref[pl.ds(k * _B, _B), :]
        lttile = jax.lax.dot_general(
            chunk, eye, (((0,), (0,)), ((), ())),
            precision=jax.lax.Precision.HIGHEST,
            preferred_element_type=jnp.float32)
        p = lttile + g[:, k * 128:(k + 1) * 128]
        ci = cidx[:, k * 128:(k + 1) * 128]
        upd = p > rv
        rv = jnp.where(upd, p, rv)
        ri = jnp.where(upd, ci, ri)
    runval[...] = rv
    runidx[...] = ri

    @pl.when(j == _NBT - 1)
    def _finish():
        rv2 = runval[...]
        ri2 = runidx[...]
        rowmax = jnp.max(rv2, axis=1, keepdims=True)
        big = jnp.int32(2**31 - 1)
        cand = jnp.where(rv2 == rowmax, ri2, big)
        idx_ref[...] = jnp.min(cand, axis=1, keepdims=True)
        val_ref[...] = rowmax


def _tc_shard(logits_t):
    return pl.pallas_call(
        _tc_body,
        grid=(_NBT,),
        in_specs=[pl.BlockSpec((_COLS, _B), lambda j: (j, 0))],
        out_specs=[
            pl.BlockSpec((_B, 1), lambda j: (0, 0)),
            pl.BlockSpec((_B, 1), lambda j: (0, 0)),
        ],
        out_shape=[
            jax.ShapeDtypeStruct((_B, 1), jnp.int32),
            jax.ShapeDtypeStruct((_B, 1), jnp.float32),
        ],
        scratch_shapes=[
            pltpu.VMEM((_B, 128), jnp.float32),
            pltpu.VMEM((_B, 128), jnp.int32),
            pltpu.VMEM((_B, _COLS), jnp.uint32),
            pltpu.VMEM((_B, _B), jnp.float32),
        ],
    )(logits_t)


def _neg_log(x):
    xb = lax.bitcast_convert_type(x, jnp.uint32)
    e = (xb >> jnp.uint32(23)).astype(jnp.int32) - jnp.int32(127)
    y = lax.bitcast_convert_type(
        (xb & jnp.uint32(0x7FFFFF)) | jnp.uint32(0x3F800000), jnp.float32)
    big = y >= jnp.float32(_SQRT2)
    y = jnp.where(big, y * jnp.float32(0.5), y)
    e = jnp.where(big, e + jnp.int32(1), e)
    z = y - jnp.float32(1.0)
    acc = jnp.full((16,), _LOG2_COEF[-1], jnp.float32)
    for c in reversed(_LOG2_COEF[:-1]):
        acc = acc * z + jnp.float32(c)
    log2x = e.astype(jnp.float32) + acc
    return log2x * jnp.float32(_NEG_LN2)


def _sc_kernel_body(lt_hbm, val_hbm, idx_hbm, buf, vout, iout, dsem):
    c = lax.axis_index("c")
    s = lax.axis_index("s")
    w = s * 2 + c
    s8 = (w * (_W8 - _WIN8)) // (_NW - 1)
    vstart = _VTC + s8 * 8
    pltpu.async_copy(lt_hbm.at[pl.ds(vstart, _WROWS)], buf, dsem).wait()

    tiny = jnp.float32(jnp.finfo(jnp.float32).tiny)
    binit = [jnp.full((16,), -jnp.inf, jnp.float32) for _ in range(8)]
    iinit = [jnp.zeros((16,), jnp.int32) for _ in range(8)]
    lane_bvs = [
        jnp.uint32(_V) * (jnp.uint32(g * 16)
                          + jax.lax.broadcasted_iota(jnp.uint32, (16,), 0))
        for g in range(8)
    ]

    def body(r, carry):
        bvs, bis = carry
        vglob = vstart + r
        cbase = vglob.astype(jnp.uint32) + jnp.uint32(_K1)
        nbvs, nbis = [], []
        for g in range(8):
            x1 = lane_bvs[g] + cbase
            bits = _threefry_bits(x1)
            fb = (bits >> jnp.uint32(9)) | jnp.uint32(0x3F800000)
            u = (lax.bitcast_convert_type(fb, jnp.float32)
                 - jnp.float32(1.0)) + tiny
            gum = _neg_log(_neg_log(u))
            phi = buf[r, pl.ds(g * 16, 16)] + gum
            upd = phi > bvs[g]
            nbvs.append(jnp.where(upd, phi, bvs[g]))
            nbis.append(jnp.where(upd, jnp.zeros((16,), jnp.int32) + vglob,
                                  bis[g]))
        return nbvs, nbis

    bvs, bis = plsc.parallel_loop(
        0, _WROWS, unroll=4, carry=(binit, iinit))(body)
    for g in range(8):
        vout[pl.ds(g * 16, 16)] = bvs[g]
        iout[pl.ds(g * 16, 16)] = bis[g]
    pltpu.sync_copy(vout, val_hbm.at[w])
    pltpu.sync_copy(iout, idx_hbm.at[w])


@functools.partial(
    pl.kernel,
    out_type=[
        jax.ShapeDtypeStruct((_NW, _B), jnp.float32),
        jax.ShapeDtypeStruct((_NW, _B), jnp.int32),
    ],
    mesh=plsc.VectorSubcoreMesh(core_axis_name="c", subcore_axis_name="s"),
    scratch_types=[
        pltpu.VMEM((_WROWS, _B), jnp.float32),
        pltpu.VMEM((_B,), jnp.float32),
        pltpu.VMEM((_B,), jnp.int32),
        pltpu.SemaphoreType.DMA,
    ],
)
def _sc_shard(lt_hbm, val_hbm, idx_hbm, buf, vout, iout, dsem):
    _sc_kernel_body(lt_hbm, val_hbm, idx_hbm, buf, vout, iout, dsem)


def _merge_body(scv_ref, sci_ref, tci_ref, tcv_ref, out_ref):
    scv = scv_ref[...].T
    sci = sci_ref[...].T
    scm = jnp.max(scv, axis=1, keepdims=True)
    big = jnp.int32(2**31 - 1)
    cand = jnp.where(scv == scm, sci, big)
    scbi = jnp.min(cand, axis=1, keepdims=True)
    take = scm > tcv_ref[...]
    out_ref[...] = jnp.where(take, scbi, tci_ref[...])


def _merge(scval, scidx, tcidx, tcval):
    return pl.pallas_call(
        _merge_body,
        out_shape=jax.ShapeDtypeStruct((_B, 1), jnp.int32),
    )(scval, scidx, tcidx, tcval)


@jax.jit
def kernel(logits):
    lt = logits.T
    scval, scidx = _sc_shard(lt)
    tcidx, tcval = _tc_shard(lt)
    out = _merge(scval, scidx, tcidx, tcval)
    return out.reshape(_B)


# speedup vs baseline: 1.2875x; 1.0571x over previous
import functools

import jax
import jax.numpy as jnp
from jax import lax
from jax.experimental import pallas as pl
from jax.experimental.pallas import tpu as pltpu
from jax.experimental.pallas import tpu_sc as plsc

_K0 = 928981903
_K1 = 3453687069
_KS = (_K0, _K1, _K0 ^ _K1 ^ 0x1BD11BDA)

_B = 128
_V = 100000
_COLS = 2048
_NBT = 40
_VTC = _NBT * _COLS
_W = _V - _VTC
_NW = 32
_W8 = _W // 8
_WIN8 = min(_W8, -(-_W8 // _NW) + 1)
_WROWS = _WIN8 * 8
_ROT = ((13, 15, 26, 6), (17, 29, 16, 24))
_NEG_LN2 = -0.6931471805599453
_LOG2_COEF = (3.81468787113981e-10, 1.442694905424571, -0.721347561295203,
              0.4809185058346664, -0.3606918278810991, 0.28774239323991246,
              -0.239137759619135, 0.2172170985382013, -0.2062543481478371,
              0.12094720695026735)
_SQRT2 = 1.4142135623730951


def _threefry_bits(x1):
    x0 = None
    for g in range(5):
        for r in _ROT[g & 1]:
            x0 = (x1 + jnp.uint32(_KS[0])) if x0 is None else (x0 + x1)
            x1 = ((x1 << jnp.uint32(r)) | (x1 >> jnp.uint32(32 - r))) ^ x0
        x0 = x0 + jnp.uint32(_KS[(g + 1) % 3])
        x1 = x1 + jnp.uint32((_KS[(g + 2) % 3] + g + 1) & 0xFFFFFFFF)
    return x0 ^ x1


def _tc_body(logits_ref, idx_ref, val_ref, runval, runidx, base_s, eye_s):
    j = pl.program_id(0)

    @pl.when(j == 0)
    def _init():
        runval[...] = jnp.full((_B, 128), -jnp.inf, jnp.float32)
        runidx[...] = jnp.zeros((_B, 128), jnp.int32)
        row = jax.lax.broadcasted_iota(jnp.uint32, (_B, _COLS), 0)
        col = jax.lax.broadcasted_iota(jnp.uint32, (_B, _COLS), 1)
        base_s[...] = row * jnp.uint32(_V) + col + jnp.uint32(_K1)
        r8 = jax.lax.broadcasted_iota(jnp.int32, (_B, _B), 0)
        c8 = jax.lax.broadcasted_iota(jnp.int32, (_B, _B), 1)
        eye_s[...] = jnp.where(r8 == c8, jnp.float32(1.0), jnp.float32(0.0))

    x1 = base_s[...] + (j * _COLS).astype(jnp.uint32)
    bits = _threefry_bits(x1)
    fb = (bits >> jnp.uint32(9)) | jnp.uint32(0x3F800000)
    tiny = jnp.float32(jnp.finfo(jnp.float32).tiny)
    u = (pltpu.bitcast(fb, jnp.float32) - jnp.float32(1.0)) + tiny
    g = -jnp.log(-jnp.log(u))
    cidx = jax.lax.broadcasted_iota(jnp.int32, (_B, _COLS), 1) + j * _COLS

    eye = eye_s[...]
    rv = runval[...]
    ri = runidx[...]
    for k in range(_COLS // 128):
        chunk = logits_ref[pl.ds(k * _B, _B), :]
        lttile = jax.lax.dot_general(
            chunk, eye, (((0,), (0,)), ((), ())),
            precision=jax.lax.Precision.HIGHEST,
            preferred_element_type=jnp.float32)
        p = lttile + g[:, k * 128:(k + 1) * 128]
        ci = cidx[:, k * 128:(k + 1) * 128]
        upd = p > rv
        rv = jnp.where(upd, p, rv)
        ri = jnp.where(upd, ci, ri)
    runval[...] = rv
    runidx[...] = ri

    @pl.when(j == _NBT - 1)
    def _finish():
        rv2 = runval[...]
        ri2 = runidx[...]
        rowmax = jnp.max(rv2, axis=1, keepdims=True)
        big = jnp.int32(2**31 - 1)
        cand = jnp.where(rv2 == rowmax, ri2, big)
        idx_ref[...] = jnp.min(cand, axis=1, keepdims=True)
        val_ref[...] = rowmax


def _tc_shard(logits_t):
    return pl.pallas_call(
        _tc_body,
        grid=(_NBT,),
        in_specs=[pl.BlockSpec((_COLS, _B), lambda j: (j, 0))],
        out_specs=[
            pl.BlockSpec((_B, 1), lambda j: (0, 0)),
            pl.BlockSpec((_B, 1), lambda j: (0, 0)),
        ],
        out_shape=[
            jax.ShapeDtypeStruct((_B, 1), jnp.int32),
            jax.ShapeDtypeStruct((_B, 1), jnp.float32),
        ],
        scratch_shapes=[
            pltpu.VMEM((_B, 128), jnp.float32),
            pltpu.VMEM((_B, 128), jnp.int32),
            pltpu.VMEM((_B, _COLS), jnp.uint32),
            pltpu.VMEM((_B, _B), jnp.float32),
        ],
    )(logits_t)


def _neg_log(x):
    xb = lax.bitcast_convert_type(x, jnp.uint32)
    e = (xb >> jnp.uint32(23)).astype(jnp.int32) - jnp.int32(127)
    y = lax.bitcast_convert_type(
        (xb & jnp.uint32(0x7FFFFF)) | jnp.uint32(0x3F800000), jnp.float32)
    big = y >= jnp.float32(_SQRT2)
    y = jnp.where(big, y * jnp.float32(0.5), y)
    e = jnp.where(big, e + jnp.int32(1), e)
    z = y - jnp.float32(1.0)
    acc = jnp.full((16,), _LOG2_COEF[-1], jnp.float32)
    for c in reversed(_LOG2_COEF[:-1]):
        acc = acc * z + jnp.float32(c)
    log2x = e.astype(jnp.float32) + acc
    return log2x * jnp.float32(_NEG_LN2)


def _sc_kernel_body(lt_hbm, val_hbm, idx_hbm, buf, vout, iout, dsem):
    c = lax.axis_index("c")
    s = lax.axis_index("s")
    w = s * 2 + c
    s8 = (w * (_W8 - _WIN8)) // (_NW - 1)
    vstart = _VTC + s8 * 8
    pltpu.async_copy(lt_hbm.at[pl.ds(vstart, _WROWS)], buf, dsem).wait()

    tiny = jnp.float32(jnp.finfo(jnp.float32).tiny)
    binit = [jnp.full((16,), -jnp.inf, jnp.float32) for _ in range(8)]
    iinit = [jnp.zeros((16,), jnp.int32) for _ in range(8)]
    lane_bvs = [
        jnp.uint32(_V) * (jnp.uint32(g * 16)
                          + jax.lax.broadcasted_iota(jnp.uint32, (16,), 0))
        for g in range(8)
    ]

    def body(r, carry):
        bvs, bis = carry
        vglob = vstart + r
        cbase = vglob.astype(jnp.uint32) + jnp.uint32(_K1)
        nbvs, nbis = [], []
        for g in range(8):
            x1 = lane_bvs[g] + cbase
            bits = _threefry_bits(x1)
            fb = (bits >> jnp.uint32(9)) | jnp.uint32(0x3F800000)
            u = (lax.bitcast_convert_type(fb, jnp.float32)
                 - jnp.float32(1.0)) + tiny
            gum = _neg_log(_neg_log(u))
            phi = buf[r, pl.ds(g * 16, 16)] + gum
            upd = phi > bvs[g]
            nbvs.append(jnp.where(upd, phi, bvs[g]))
            nbis.append(jnp.where(upd, jnp.zeros((16,), jnp.int32) + vglob,
                                  bis[g]))
        return nbvs, nbis

    bvs, bis = plsc.parallel_loop(
        0, _WROWS, unroll=4, carry=(binit, iinit))(body)
    for g in range(8):
        vout[pl.ds(g * 16, 16)] = bvs[g]
        iout[pl.ds(g * 16, 16)] = bis[g]
    pltpu.sync_copy(vout, val_hbm.at[w])
    pltpu.sync_copy(iout, idx_hbm.at[w])


@functools.partial(
    pl.kernel,
    out_type=[
        jax.ShapeDtypeStruct((_NW, _B), jnp.float32),
        jax.ShapeDtypeStruct((_NW, _B), jnp.int32),
    ],
    mesh=plsc.VectorSubcoreMesh(core_axis_name="c", subcore_axis_name="s"),
    scratch_types=[
        pltpu.VMEM((_WROWS, _B), jnp.float32),
        pltpu.VMEM((_B,), jnp.float32),
        pltpu.VMEM((_B,), jnp.int32),
        pltpu.SemaphoreType.DMA,
    ],
)
def _sc_shard(lt_hbm, val_hbm, idx_hbm, buf, vout, iout, dsem):
    _sc_kernel_body(lt_hbm, val_hbm, idx_hbm, buf, vout, iout, dsem)


def _merge_body(scv_ref, sci_ref, tci_ref, tcv_ref, out_ref):
    scv = scv_ref[...].T
    sci = sci_ref[...].T
    scm = jnp.max(scv, axis=1, keepdims=True)
    big = jnp.int32(2**31 - 1)
    cand = jnp.where(scv == scm, sci, big)
    scbi = jnp.min(cand, axis=1, keepdims=True)
    take = scm > tcv_ref[...]
    out_ref[...] = jnp.where(take, scbi, tci_ref[...])


def _merge(scval, scidx, tcidx, tcval):
    return pl.pallas_call(
        _merge_body,
        out_shape=jax.ShapeDtypeStruct((_B, 1), jnp.int32),
    )(scval, scidx, tcidx, tcval)


@jax.jit
def kernel(logits):
    lt = logits.T
    scval, scidx = _sc_shard(lt)
    tcidx, tcval = _tc_shard(lt)
    out = _merge(scval, scidx, tcidx, tcval)
    return out.reshape(_B)


# speedup vs baseline: 1.3111x; 1.0183x over previous
import functools

import jax
import jax.numpy as jnp
from jax import lax
from jax.experimental import pallas as pl
from jax.experimental.pallas import tpu as pltpu
from jax.experimental.pallas import tpu_sc as plsc

_K0 = 928981903
_K1 = 3453687069
_KS = (_K0, _K1, _K0 ^ _K1 ^ 0x1BD11BDA)

_B = 128
_V = 100000
_COLS = 2048
_NBT = 39
_VTC = _NBT * _COLS
_W = _V - _VTC
_NW = 32
_W8 = _W // 8
_WIN8 = min(_W8, -(-_W8 // _NW) + 1)
_WROWS = _WIN8 * 8
_ROT = ((13, 15, 26, 6), (17, 29, 16, 24))
_NEG_LN2 = -0.6931471805599453
_LOG2_COEF = (3.81468787113981e-10, 1.442694905424571, -0.721347561295203,
              0.4809185058346664, -0.3606918278810991, 0.28774239323991246,
              -0.239137759619135, 0.2172170985382013, -0.2062543481478371,
              0.12094720695026735)
_SQRT2 = 1.4142135623730951


def _threefry_bits(x1):
    x0 = None
    for g in range(5):
        for r in _ROT[g & 1]:
            x0 = (x1 + jnp.uint32(_KS[0])) if x0 is None else (x0 + x1)
            x1 = ((x1 << jnp.uint32(r)) | (x1 >> jnp.uint32(32 - r))) ^ x0
        x0 = x0 + jnp.uint32(_KS[(g + 1) % 3])
        x1 = x1 + jnp.uint32((_KS[(g + 2) % 3] + g + 1) & 0xFFFFFFFF)
    return x0 ^ x1


def _tc_body(logits_ref, idx_ref, val_ref, runval, runidx, base_s, eye_s):
    j = pl.program_id(0)

    @pl.when(j == 0)
    def _init():
        runval[...] = jnp.full((_B, 128), -jnp.inf, jnp.float32)
        runidx[...] = jnp.zeros((_B, 128), jnp.int32)
        row = jax.lax.broadcasted_iota(jnp.uint32, (_B, _COLS), 0)
        col = jax.lax.broadcasted_iota(jnp.uint32, (_B, _COLS), 1)
        base_s[...] = row * jnp.uint32(_V) + col + jnp.uint32(_K1)
        r8 = jax.lax.broadcasted_iota(jnp.int32, (_B, _B), 0)
        c8 = jax.lax.broadcasted_iota(jnp.int32, (_B, _B), 1)
        eye_s[...] = jnp.where(r8 == c8, jnp.float32(1.0), jnp.float32(0.0))

    x1 = base_s[...] + (j * _COLS).astype(jnp.uint32)
    bits = _threefry_bits(x1)
    fb = (bits >> jnp.uint32(9)) | jnp.uint32(0x3F800000)
    tiny = jnp.float32(jnp.finfo(jnp.float32).tiny)
    u = (pltpu.bitcast(fb, jnp.float32) - jnp.float32(1.0)) + tiny
    g = -jnp.log(-jnp.log(u))
    cidx = jax.lax.broadcasted_iota(jnp.int32, (_B, _COLS), 1) + j * _COLS

    eye = eye_s[...]
    rv = runval[...]
    ri = runidx[...]
    for k in range(_COLS // 128):
        chunk = logits_ref[pl.ds(k * _B, _B), :]
        lttile = jax.lax.dot_general(
            chunk, eye, (((0,), (0,)), ((), ())),
            precision=jax.lax.Precision.HIGHEST,
            preferred_element_type=jnp.float32)
        p = lttile + g[:, k * 128:(k + 1) * 128]
        ci = cidx[:, k * 128:(k + 1) * 128]
        upd = p > rv
        rv = jnp.where(upd, p, rv)
        ri = jnp.where(upd, ci, ri)
    runval[...] = rv
    runidx[...] = ri

    @pl.when(j == _NBT - 1)
    def _finish():
        rv2 = runval[...]
        ri2 = runidx[...]
        rowmax = jnp.max(rv2, axis=1, keepdims=True)
        big = jnp.int32(2**31 - 1)
        cand = jnp.where(rv2 == rowmax, ri2, big)
        idx_ref[...] = jnp.min(cand, axis=1, keepdims=True)
        val_ref[...] = rowmax


def _tc_shard(logits_t):
    return pl.pallas_call(
        _tc_body,
        grid=(_NBT,),
        in_specs=[pl.BlockSpec((_COLS, _B), lambda j: (j, 0))],
        out_specs=[
            pl.BlockSpec((_B, 1), lambda j: (0, 0)),
            pl.BlockSpec((_B, 1), lambda j: (0, 0)),
        ],
        out_shape=[
            jax.ShapeDtypeStruct((_B, 1), jnp.int32),
            jax.ShapeDtypeStruct((_B, 1), jnp.float32),
        ],
        scratch_shapes=[
            pltpu.VMEM((_B, 128), jnp.float32),
            pltpu.VMEM((_B, 128), jnp.int32),
            pltpu.VMEM((_B, _COLS), jnp.uint32),
            pltpu.VMEM((_B, _B), jnp.float32),
        ],
    )(logits_t)


def _neg_log(x):
    xb = lax.bitcast_convert_type(x, jnp.uint32)
    e = (xb >> jnp.uint32(23)).astype(jnp.int32) - jnp.int32(127)
    y = lax.bitcast_convert_type(
        (xb & jnp.uint32(0x7FFFFF)) | jnp.uint32(0x3F800000), jnp.float32)
    big = y >= jnp.float32(_SQRT2)
    y = jnp.where(big, y * jnp.float32(0.5), y)
    e = jnp.where(big, e + jnp.int32(1), e)
    z = y - jnp.float32(1.0)
    acc = jnp.full((16,), _LOG2_COEF[-1], jnp.float32)
    for c in reversed(_LOG2_COEF[:-1]):
        acc = acc * z + jnp.float32(c)
    log2x = e.astype(jnp.float32) + acc
    return log2x * jnp.float32(_NEG_LN2)


def _sc_kernel_body(lt_hbm, val_hbm, idx_hbm, buf, vout, iout, dsem):
    c = lax.axis_index("c")
    s = lax.axis_index("s")
    w = s * 2 + c
    s8 = (w * (_W8 - _WIN8)) // (_NW - 1)
    vstart = _VTC + s8 * 8
    pltpu.async_copy(lt_hbm.at[pl.ds(vstart, _WROWS)], buf, dsem).wait()

    tiny = jnp.float32(jnp.finfo(jnp.float32).tiny)
    binit = [jnp.full((16,), -jnp.inf, jnp.float32) for _ in range(8)]
    iinit = [jnp.zeros((16,), jnp.int32) for _ in range(8)]
    lane_bvs = [
        jnp.uint32(_V) * (jnp.uint32(g * 16)
                          + jax.lax.broadcasted_iota(jnp.uint32, (16,), 0))
        for g in range(8)
    ]

    def body(r, carry):
        bvs, bis = carry
        vglob = vstart + r
        cbase = vglob.astype(jnp.uint32) + jnp.uint32(_K1)
        nbvs, nbis = [], []
        for g in range(8):
            x1 = lane_bvs[g] + cbase
            bits = _threefry_bits(x1)
            fb = (bits >> jnp.uint32(9)) | jnp.uint32(0x3F800000)
            u = (lax.bitcast_convert_type(fb, jnp.float32)
                 - jnp.float32(1.0)) + tiny
            gum = _neg_log(_neg_log(u))
            phi = buf[r, pl.ds(g * 16, 16)] + gum
            upd = phi > bvs[g]
            nbvs.append(jnp.where(upd, phi, bvs[g]))
            nbis.append(jnp.where(upd, jnp.zeros((16,), jnp.int32) + vglob,
                                  bis[g]))
        return nbvs, nbis

    bvs, bis = plsc.parallel_loop(
        0, _WROWS, unroll=4, carry=(binit, iinit))(body)
    for g in range(8):
        vout[pl.ds(g * 16, 16)] = bvs[g]
        iout[pl.ds(g * 16, 16)] = bis[g]
    pltpu.sync_copy(vout, val_hbm.at[w])
    pltpu.sync_copy(iout, idx_hbm.at[w])


@functools.partial(
    pl.kernel,
    out_type=[
        jax.ShapeDtypeStruct((_NW, _B), jnp.float32),
        jax.ShapeDtypeStruct((_NW, _B), jnp.int32),
    ],
    mesh=plsc.VectorSubcoreMesh(core_axis_name="c", subcore_axis_name="s"),
    scratch_types=[
        pltpu.VMEM((_WROWS, _B), jnp.float32),
        pltpu.VMEM((_B,), jnp.float32),
        pltpu.VMEM((_B,), jnp.int32),
        pltpu.SemaphoreType.DMA,
    ],
)
def _sc_shard(lt_hbm, val_hbm, idx_hbm, buf, vout, iout, dsem):
    _sc_kernel_body(lt_hbm, val_hbm, idx_hbm, buf, vout, iout, dsem)


def _merge_body(scv_ref, sci_ref, tci_ref, tcv_ref, out_ref):
    scv = scv_ref[...].T
    sci = sci_ref[...].T
    scm = jnp.max(scv, axis=1, keepdims=True)
    big = jnp.int32(2**31 - 1)
    cand = jnp.where(scv == scm, sci, big)
    scbi = jnp.min(cand, axis=1, keepdims=True)
    take = scm > tcv_ref[...]
    out_ref[...] = jnp.where(take, scbi, tci_ref[...])


def _merge(scval, scidx, tcidx, tcval):
    return pl.pallas_call(
        _merge_body,
        out_shape=jax.ShapeDtypeStruct((_B, 1), jnp.int32),
    )(scval, scidx, tcidx, tcval)


@jax.jit
def kernel(logits):
    lt = logits.T
    scval, scidx = _sc_shard(lt)
    tcidx, tcval = _tc_shard(lt)
    out = _merge(scval, scidx, tcidx, tcval)
    return out.reshape(_B)
